# baseline (speedup 1.0000x reference)
import functools

import jax
import jax.numpy as jnp
from jax import lax
from jax.experimental import pallas as pl
from jax.experimental.pallas import tpu as pltpu
from jax.experimental.pallas import tpu_sc as plsc

N = 10000
E = 320000
D = 128
S = 256
M = 32
V = 3
TEMP = 0.5

NC = 2
NS = 16
NW = NC * NS

EB = 128
ENB = 80
EPAD = NW * ENB * EB
NP = 10240

PB = 128
PNB = 2


def _zero_fill(ref, rows):
    def body(i, carry):
        r = i // (D // 16)
        c = (i % (D // 16)) * 16
        ref[r, pl.ds(c, 16)] = jnp.zeros((16,), jnp.float32)
        return carry
    lax.fori_loop(0, rows * (D // 16), body, 0)


@functools.lru_cache(maxsize=None)
def _make_segsum(num_seg, nb, b, nb0=None):
    if nb0 is None:
        nb0 = nb
    nb1 = 2 * nb - nb0
    assert nb0 % 2 == 0 and nb1 % 2 == 0
    zr = num_seg // NS
    zb = zr if zr <= 64 else 64
    assert zr % zb == 0 and zr % 8 == 0
    mesh = plsc.VectorSubcoreMesh(core_axis_name="c", subcore_axis_name="s",
                                  num_cores=NC, num_subcores=NS)

    @functools.partial(
        pl.kernel,
        out_type=jax.ShapeDtypeStruct((NC * num_seg, D), jnp.float32),
        mesh=mesh,
        scratch_types=[
            pltpu.VMEM((2, 2, b), jnp.int32),
            pltpu.VMEM((2, b, D), jnp.float32),
            pltpu.VMEM((zb, D), jnp.float32),
            pltpu.VMEM_SHARED((num_seg, D), jnp.float32),
            pltpu.SemaphoreType.DMA,
            pltpu.SemaphoreType.DMA,
            pltpu.SemaphoreType.DMA,
            pltpu.SemaphoreType.DMA,
            pltpu.SemaphoreType.DMA,
            pltpu.SemaphoreType.DMA,
        ],
    )
    def segsum(table_hbm, sd_hbm, out_hbm, sd, rows, zbuf, acc,
               g0, g1, i0, i1, s0, s1):
        c = lax.axis_index("c")
        s = lax.axis_index("s")
        nb = lax.select(c == 0, nb0, nb1)
        q0 = lax.select(c == 0, s * nb0, NS * nb0 + s * nb1)

        def work():

            pltpu.async_copy(sd_hbm.at[q0], sd.at[0], i0)
            pltpu.async_copy(sd_hbm.at[q0 + 1], sd.at[1], i1)

            _zero_fill(zbuf, zb)
            row0 = pl.multiple_of(s * zr, 8)
            for q in range(zr // zb):
                pltpu.async_copy(zbuf, acc.at[pl.ds(row0 + q * zb, zb)], g1)
            for q in range(zr // zb):
                pltpu.make_async_copy(zbuf, acc.at[pl.ds(row0, zb)], g1).wait()

            pltpu.make_async_copy(sd_hbm.at[q0], sd.at[0], i0).wait()
            pltpu.async_copy(table_hbm.at[sd.at[0, 0]], rows.at[0], g0)
            plsc.subcore_barrier()

            def body(i, carry):
                k0 = 2 * i

                pltpu.make_async_copy(sd_hbm.at[q0 + 1], sd.at[1], i1).wait()

                @pl.when(k0 > 0)
                def _():
                    pltpu.make_async_copy(
                        rows.at[1], acc.at[sd.at[1, 1]], s1).wait()

                pltpu.async_copy(table_hbm.at[sd.at[1, 0]], rows.at[1], g1)
                pltpu.make_async_copy(
                    table_hbm.at[sd.at[0, 0]], rows.at[0], g0).wait()
                pltpu.async_copy(rows.at[0], acc.at[sd.at[0, 1]], s0, add=True)

                @pl.when(k0 + 2 < nb)
                def _():
                    pltpu.async_copy(sd_hbm.at[q0 + k0 + 2], sd.at[0], i0)

                pltpu.make_async_copy(
                    table_hbm.at[sd.at[1, 0]], rows.at[1], g1).wait()
                pltpu.async_copy(rows.at[1], acc.at[sd.at[1, 1]], s1, add=True)

                @pl.when(k0 + 3 < nb)
                def _():
                    pltpu.async_copy(sd_hbm.at[q0 + k0 + 3], sd.at[1], i1)

                @pl.when(k0 + 2 < nb)
                def _():
                    pltpu.make_async_copy(
                        sd_hbm.at[q0 + k0 + 2], sd.at[0], i0).wait()
                    pltpu.make_async_copy(
                        rows.at[0], acc.at[sd.at[0, 1]], s0).wait()
                    pltpu.async_copy(table_hbm.at[sd.at[0, 0]], rows.at[0], g0)

                return carry

            lax.fori_loop(0, nb // 2, body, 0)
            pltpu.make_async_copy(rows.at[0], acc.at[sd.at[0, 1]], s0).wait()
            pltpu.make_async_copy(rows.at[1], acc.at[sd.at[1, 1]], s1).wait()
            plsc.subcore_barrier()

            base = pl.multiple_of(c * num_seg + row0, 8)
            pltpu.sync_copy(acc.at[pl.ds(row0, zr)], out_hbm.at[pl.ds(base, zr)])

        work()

    return segsum


@functools.lru_cache(maxsize=None)
def _make_cf_gather():
    bpw = (4 * S) // NW
    mesh = plsc.VectorSubcoreMesh(core_axis_name="c", subcore_axis_name="s",
                                  num_cores=NC, num_subcores=NS)

    @functools.partial(
        pl.kernel,
        out_type=jax.ShapeDtypeStruct((4 * S, D), jnp.float32),
        mesh=mesh,
        scratch_types=[
            pltpu.VMEM((bpw,), jnp.int32),
            pltpu.VMEM((bpw, D), jnp.float32),
            pltpu.VMEM((bpw, D), jnp.float32),
            pltpu.SemaphoreType.DMA,
            pltpu.SemaphoreType.DMA,
        ],
    )
    def cfgather(p0_hbm, p1_hbm, idx_hbm, out_hbm, idxw, ra, rb, g0, g1):
        c = lax.axis_index("c")
        s = lax.axis_index("s")
        wid = c * NS + s
        pltpu.sync_copy(idx_hbm.at[wid], idxw)
        cp0 = pltpu.async_copy(p0_hbm.at[idxw], ra, g0)
        cp1 = pltpu.async_copy(p1_hbm.at[idxw], rb, g1)
        cp0.wait()
        cp1.wait()

        def body(i, carry):
            r = i // (D // 16)
            col = (i % (D // 16)) * 16
            va = ra[r, pl.ds(col, 16)]
            vb = rb[r, pl.ds(col, 16)]
            ra[r, pl.ds(col, 16)] = (va + vb) * (1.0 / M)
            return carry

        lax.fori_loop(0, bpw * (D // 16), body, 0)
        pltpu.sync_copy(ra, out_hbm.at[pl.ds(wid * bpw, bpw)])

    return cfgather


BN = 1000


def _mlp_body(norm, p_ref, wa_ref, ba_ref, wb_ref, bb_ref, o_ref):
    x = p_ref[0]
    for q in range(1, p_ref.shape[0]):
        x = x + p_ref[q]
    y = jnp.maximum(
        jnp.dot(x, wa_ref[...], preferred_element_type=jnp.float32) + ba_ref[...], 0.0)
    z = jnp.maximum(
        jnp.dot(y, wb_ref[...], preferred_element_type=jnp.float32) + bb_ref[...], 0.0)
    if norm:
        nrm = jnp.sqrt(jnp.sum(z * z, axis=1, keepdims=True))
        z = z / jnp.maximum(nrm, 1e-12)
    o_ref[...] = z


def _mlp(p, wa, ba, wb, bb, norm):
    return pl.pallas_call(
        functools.partial(_mlp_body, norm),
        out_shape=jax.ShapeDtypeStruct((N, D), jnp.float32),
        grid=(N // BN,),
        in_specs=[
            pl.BlockSpec((p.shape[0], BN, D), lambda i: (0, i, 0)),
            pl.BlockSpec((D, D), lambda i: (0, 0)),
            pl.BlockSpec((1, D), lambda i: (0, 0)),
            pl.BlockSpec((D, D), lambda i: (0, 0)),
            pl.BlockSpec((1, D), lambda i: (0, 0)),
        ],
        out_specs=pl.BlockSpec((BN, D), lambda i: (i, 0)),
    )(p, wa, ba, wb, bb)


def _loss_body(cf_ref, o_ref):
    i = pl.program_id(0)
    f = cf_ref[0]
    nrm = jnp.sqrt(jnp.sum(f * f, axis=1, keepdims=True))
    fn = f / jnp.maximum(nrm, 1e-12)
    logits = lax.dot_general(
        fn, fn, (((1,), (1,)), ((), ())), preferred_element_type=jnp.float32)
    logits = logits * (1.0 / TEMP)
    logits = logits - jnp.max(logits, axis=1, keepdims=True)
    r = lax.broadcasted_iota(jnp.int32, logits.shape, 0)
    cc = lax.broadcasted_iota(jnp.int32, logits.shape, 1)
    lmask = jnp.where(r == cc, 0.0, 1.0)
    pmask = jnp.where(lax.rem(r, 64) == lax.rem(cc, 64), 1.0, 0.0) * lmask
    el = jnp.exp(logits) * lmask
    denom = jnp.sum(el, axis=1, keepdims=True) + 1e-12
    lp = logits - jnp.log(denom)
    mlpp = jnp.sum(pmask * lp, axis=1) / jnp.sum(pmask, axis=1)
    bl = -jnp.sum(mlpp) / logits.shape[0]

    @pl.when(i == 0)
    def _():
        o_ref[...] = jnp.zeros((1, 1), jnp.float32)

    o_ref[...] += bl.reshape(1, 1)


def _loss(cf4):
    nblk = cf4.shape[0]
    return pl.pallas_call(
        _loss_body,
        out_shape=jax.ShapeDtypeStruct((1, 1), jnp.float32),
        grid=(nblk,),
        in_specs=[pl.BlockSpec((1, cf4.shape[1], D), lambda i: (i, 0, 0))],
        out_specs=pl.BlockSpec((1, 1), lambda i: (0, 0)),
    )(cf4)


def kernel(seq1, adj, train_nodeSet, pathDict, bsz,
           W0a, b0a, W0b, b0b, W1a, b1a, W1b, b1b, W2a, b2a, W2b, b2b):
    npad = EPAD - E
    pad_src = jnp.arange(npad, dtype=jnp.int32) % N
    pad_dst = N + (jnp.arange(npad, dtype=jnp.int32) % 128)
    srcp = jnp.concatenate([adj[0], pad_src])
    dstp = jnp.concatenate([adj[1], pad_dst])
    sd_edges = jnp.stack([srcp.reshape(-1, EB), dstp.reshape(-1, EB)], axis=1)

    _segsum_edges = _make_segsum(NP, ENB, EB)
    _segsum_pool = _make_segsum(S, PNB, PB)
    _cf_gather = _make_cf_gather()

    h = seq1
    layers = ((W0a, b0a, W0b, b0b), (W1a, b1a, W1b, b1b), (W2a, b2a, W2b, b2b))
    for li, (wa, ba, wb, bb) in enumerate(layers):
        p = _segsum_edges(h, sd_edges).reshape(NC, NP, D)[:, :N]
        h = _mlp(p, wa, ba.reshape(1, D), wb, bb.reshape(1, D),
                 norm=(li == len(layers) - 1))

    sd_pool = jnp.stack(
        [train_nodeSet.reshape(-1, PB),
         jnp.repeat(jnp.arange(S, dtype=jnp.int32), M).reshape(-1, PB)], axis=1)
    psub = _segsum_pool(h, sd_pool)

    vi = jnp.concatenate(
        [pathDict, jnp.arange(S, dtype=pathDict.dtype)[:, None]], axis=1)
    cfidx = jnp.transpose(vi.reshape(4, 64, V + 1), (0, 2, 1)).reshape(NW, -1)
    cfidx = cfidx.astype(jnp.int32)

    cf = _cf_gather(psub[:S], psub[S:], cfidx)
    lsum = _loss(cf.reshape(4, (V + 1) * 64, D))

    nb = S // bsz
    return lsum[0, 0] / nb


# speedup vs baseline: 1.0352x; 1.0352x over previous
import functools

import jax
import jax.numpy as jnp
from jax import lax
from jax.experimental import pallas as pl
from jax.experimental.pallas import tpu as pltpu
from jax.experimental.pallas import tpu_sc as plsc

N = 10000
E = 320000
D = 128
S = 256
M = 32
V = 3
TEMP = 0.5

NC = 2
NS = 16
NW = NC * NS

EB = 128
ENB = 80
EPAD = NW * ENB * EB
NP = 10240

PB = 128
PNB = 2


def _zero_fill(ref, rows):
    def body(i, carry):
        r = i // (D // 16)
        c = (i % (D // 16)) * 16
        ref[r, pl.ds(c, 16)] = jnp.zeros((16,), jnp.float32)
        return carry
    lax.fori_loop(0, rows * (D // 16), body, 0)


@functools.lru_cache(maxsize=None)
def _make_segsum(num_seg, nb, b, nb0=None):
    if nb0 is None:
        nb0 = nb
    nb1 = 2 * nb - nb0
    assert nb0 % 2 == 0 and nb1 % 2 == 0
    zr = num_seg // NS
    zb = zr if zr <= 64 else 64
    assert zr % zb == 0 and zr % 8 == 0
    mesh = plsc.VectorSubcoreMesh(core_axis_name="c", subcore_axis_name="s",
                                  num_cores=NC, num_subcores=NS)

    @functools.partial(
        pl.kernel,
        out_type=jax.ShapeDtypeStruct((NC * num_seg, D), jnp.float32),
        mesh=mesh,
        scratch_types=[
            pltpu.VMEM((2, 2, b), jnp.int32),
            pltpu.VMEM((2, b, D), jnp.float32),
            pltpu.VMEM((zb, D), jnp.float32),
            pltpu.VMEM_SHARED((num_seg, D), jnp.float32),
            pltpu.SemaphoreType.DMA,
            pltpu.SemaphoreType.DMA,
            pltpu.SemaphoreType.DMA,
            pltpu.SemaphoreType.DMA,
        ],
    )
    def segsum(table_hbm, sd_hbm, out_hbm, sd, rows, zbuf, acc, g0, g1, i0, i1):
        c = lax.axis_index("c")
        s = lax.axis_index("s")
        nb = lax.select(c == 0, nb0, nb1)
        q0 = lax.select(c == 0, s * nb0, NS * nb0 + s * nb1)

        def work():

            pltpu.async_copy(sd_hbm.at[q0], sd.at[0], i0)
            pltpu.async_copy(sd_hbm.at[q0 + 1], sd.at[1], i1)

            _zero_fill(zbuf, zb)
            row0 = pl.multiple_of(s * zr, 8)
            for q in range(zr // zb):
                pltpu.async_copy(zbuf, acc.at[pl.ds(row0 + q * zb, zb)], g1)
            for q in range(zr // zb):
                pltpu.make_async_copy(zbuf, acc.at[pl.ds(row0, zb)], g1).wait()

            pltpu.make_async_copy(sd_hbm.at[q0], sd.at[0], i0).wait()
            pltpu.async_copy(table_hbm.at[sd.at[0, 0]], rows.at[0], g0)
            plsc.subcore_barrier()

            def body(i, carry):
                k0 = 2 * i
                pltpu.make_async_copy(sd_hbm.at[q0 + 1], sd.at[1], i1).wait()
                pltpu.async_copy(table_hbm.at[sd.at[1, 0]], rows.at[1], g1)
                pltpu.make_async_copy(
                    table_hbm.at[sd.at[0, 0]], rows.at[0], g0).wait()
                pltpu.sync_copy(rows.at[0], acc.at[sd.at[0, 1]], add=True)

                @pl.when(k0 + 2 < nb)
                def _():
                    pltpu.async_copy(sd_hbm.at[q0 + k0 + 2], sd.at[0], i0)

                pltpu.make_async_copy(
                    table_hbm.at[sd.at[1, 0]], rows.at[1], g1).wait()
                pltpu.sync_copy(rows.at[1], acc.at[sd.at[1, 1]], add=True)

                @pl.when(k0 + 3 < nb)
                def _():
                    pltpu.async_copy(sd_hbm.at[q0 + k0 + 3], sd.at[1], i1)

                @pl.when(k0 + 2 < nb)
                def _():
                    pltpu.make_async_copy(
                        sd_hbm.at[q0 + k0 + 2], sd.at[0], i0).wait()
                    pltpu.async_copy(table_hbm.at[sd.at[0, 0]], rows.at[0], g0)

                return carry

            lax.fori_loop(0, nb // 2, body, 0)
            plsc.subcore_barrier()

            base = pl.multiple_of(c * num_seg + row0, 8)
            pltpu.sync_copy(acc.at[pl.ds(row0, zr)], out_hbm.at[pl.ds(base, zr)])

        work()

    return segsum


@functools.lru_cache(maxsize=None)
def _make_cf_gather():
    bpw = (4 * S) // NW
    mesh = plsc.VectorSubcoreMesh(core_axis_name="c", subcore_axis_name="s",
                                  num_cores=NC, num_subcores=NS)

    @functools.partial(
        pl.kernel,
        out_type=jax.ShapeDtypeStruct((4 * S, D), jnp.float32),
        mesh=mesh,
        scratch_types=[
            pltpu.VMEM((bpw,), jnp.int32),
            pltpu.VMEM((bpw, D), jnp.float32),
            pltpu.VMEM((bpw, D), jnp.float32),
            pltpu.SemaphoreType.DMA,
            pltpu.SemaphoreType.DMA,
        ],
    )
    def cfgather(p0_hbm, p1_hbm, idx_hbm, out_hbm, idxw, ra, rb, g0, g1):
        c = lax.axis_index("c")
        s = lax.axis_index("s")
        wid = c * NS + s
        pltpu.sync_copy(idx_hbm.at[wid], idxw)
        cp0 = pltpu.async_copy(p0_hbm.at[idxw], ra, g0)
        cp1 = pltpu.async_copy(p1_hbm.at[idxw], rb, g1)
        cp0.wait()
        cp1.wait()

        def body(i, carry):
            r = i // (D // 16)
            col = (i % (D // 16)) * 16
            va = ra[r, pl.ds(col, 16)]
            vb = rb[r, pl.ds(col, 16)]
            ra[r, pl.ds(col, 16)] = (va + vb) * (1.0 / M)
            return carry

        lax.fori_loop(0, bpw * (D // 16), body, 0)
        pltpu.sync_copy(ra, out_hbm.at[pl.ds(wid * bpw, bpw)])

    return cfgather


BN = 1000


def _mlp_body(norm, p_ref, wa_ref, ba_ref, wb_ref, bb_ref, o_ref):
    x = p_ref[0]
    for q in range(1, p_ref.shape[0]):
        x = x + p_ref[q]
    y = jnp.maximum(
        jnp.dot(x, wa_ref[...], preferred_element_type=jnp.float32) + ba_ref[...], 0.0)
    z = jnp.maximum(
        jnp.dot(y, wb_ref[...], preferred_element_type=jnp.float32) + bb_ref[...], 0.0)
    if norm:
        nrm = jnp.sqrt(jnp.sum(z * z, axis=1, keepdims=True))
        z = z / jnp.maximum(nrm, 1e-12)
    o_ref[...] = z


def _mlp(p, wa, ba, wb, bb, norm):
    return pl.pallas_call(
        functools.partial(_mlp_body, norm),
        out_shape=jax.ShapeDtypeStruct((N, D), jnp.float32),
        grid=(N // BN,),
        in_specs=[
            pl.BlockSpec((p.shape[0], BN, D), lambda i: (0, i, 0)),
            pl.BlockSpec((D, D), lambda i: (0, 0)),
            pl.BlockSpec((1, D), lambda i: (0, 0)),
            pl.BlockSpec((D, D), lambda i: (0, 0)),
            pl.BlockSpec((1, D), lambda i: (0, 0)),
        ],
        out_specs=pl.BlockSpec((BN, D), lambda i: (i, 0)),
    )(p, wa, ba, wb, bb)


def _loss_body(cf_ref, o_ref):
    i = pl.program_id(0)
    f = cf_ref[0]
    nrm = jnp.sqrt(jnp.sum(f * f, axis=1, keepdims=True))
    fn = f / jnp.maximum(nrm, 1e-12)
    logits = lax.dot_general(
        fn, fn, (((1,), (1,)), ((), ())), preferred_element_type=jnp.float32)
    logits = logits * (1.0 / TEMP)
    logits = logits - jnp.max(logits, axis=1, keepdims=True)
    r = lax.broadcasted_iota(jnp.int32, logits.shape, 0)
    cc = lax.broadcasted_iota(jnp.int32, logits.shape, 1)
    lmask = jnp.where(r == cc, 0.0, 1.0)
    pmask = jnp.where(lax.rem(r, 64) == lax.rem(cc, 64), 1.0, 0.0) * lmask
    el = jnp.exp(logits) * lmask
    denom = jnp.sum(el, axis=1, keepdims=True) + 1e-12
    lp = logits - jnp.log(denom)
    mlpp = jnp.sum(pmask * lp, axis=1) / jnp.sum(pmask, axis=1)
    bl = -jnp.sum(mlpp) / logits.shape[0]

    @pl.when(i == 0)
    def _():
        o_ref[...] = jnp.zeros((1, 1), jnp.float32)

    o_ref[...] += bl.reshape(1, 1)


def _loss(cf4):
    nblk = cf4.shape[0]
    return pl.pallas_call(
        _loss_body,
        out_shape=jax.ShapeDtypeStruct((1, 1), jnp.float32),
        grid=(nblk,),
        in_specs=[pl.BlockSpec((1, cf4.shape[1], D), lambda i: (i, 0, 0))],
        out_specs=pl.BlockSpec((1, 1), lambda i: (0, 0)),
    )(cf4)


def kernel(seq1, adj, train_nodeSet, pathDict, bsz,
           W0a, b0a, W0b, b0b, W1a, b1a, W1b, b1b, W2a, b2a, W2b, b2b):
    npad = EPAD - E
    pad_src = jnp.arange(npad, dtype=jnp.int32) % N
    pad_dst = N + (jnp.arange(npad, dtype=jnp.int32) % 128)
    srcp = jnp.concatenate([adj[0], pad_src])
    dstp = jnp.concatenate([adj[1], pad_dst])
    sd_edges = jnp.stack([srcp.reshape(-1, EB), dstp.reshape(-1, EB)], axis=1)

    _segsum_edges = _make_segsum(NP, ENB, EB)
    _segsum_pool = _make_segsum(S, PNB, PB)
    _cf_gather = _make_cf_gather()

    h = seq1
    layers = ((W0a, b0a, W0b, b0b), (W1a, b1a, W1b, b1b), (W2a, b2a, W2b, b2b))
    for li, (wa, ba, wb, bb) in enumerate(layers):
        p = _segsum_edges(h, sd_edges).reshape(NC, NP, D)[:, :N]
        h = _mlp(p, wa, ba.reshape(1, D), wb, bb.reshape(1, D),
                 norm=(li == len(layers) - 1))

    sd_pool = jnp.stack(
        [train_nodeSet.reshape(-1, PB),
         jnp.repeat(jnp.arange(S, dtype=jnp.int32), M).reshape(-1, PB)], axis=1)
    psub = _segsum_pool(h, sd_pool)

    vi = jnp.concatenate(
        [pathDict, jnp.arange(S, dtype=pathDict.dtype)[:, None]], axis=1)
    cfidx = jnp.transpose(vi.reshape(4, 64, V + 1), (0, 2, 1)).reshape(NW, -1)
    cfidx = cfidx.astype(jnp.int32)

    cf = _cf_gather(psub[:S], psub[S:], cfidx)
    lsum = _loss(cf.reshape(4, (V + 1) * 64, D))

    nb = S // bsz
    return lsum[0, 0] / nb


# speedup vs baseline: 1.0433x; 1.0079x over previous
import functools

import jax
import jax.numpy as jnp
from jax import lax
from jax.experimental import pallas as pl
from jax.experimental.pallas import tpu as pltpu
from jax.experimental.pallas import tpu_sc as plsc

N = 10000
E = 320000
D = 128
S = 256
M = 32
V = 3
TEMP = 0.5

NC = 2
NS = 16
NW = NC * NS

EB = 128
ENB = 80
EPAD = NW * ENB * EB
NP = 10240

PB = 128
PNB = 2


def _zero_fill(ref, rows):
    def body(i, carry):
        r = i // (D // 16)
        c = (i % (D // 16)) * 16
        ref[r, pl.ds(c, 16)] = jnp.zeros((16,), jnp.float32)
        return carry
    lax.fori_loop(0, rows * (D // 16), body, 0)


@functools.lru_cache(maxsize=None)
def _make_segsum(num_seg, nb, b, nb0=None):
    if nb0 is None:
        nb0 = nb
    nb1 = 2 * nb - nb0
    assert nb0 % 2 == 0 and nb1 % 2 == 0
    zr = num_seg // NS
    zb = zr if zr <= 64 else 64
    assert zr % zb == 0 and zr % 8 == 0
    mesh = plsc.VectorSubcoreMesh(core_axis_name="c", subcore_axis_name="s",
                                  num_cores=NC, num_subcores=NS)

    @functools.partial(
        pl.kernel,
        out_type=jax.ShapeDtypeStruct((NC * num_seg, D), jnp.float32),
        mesh=mesh,
        scratch_types=[
            pltpu.VMEM((2, 2, b), jnp.int32),
            pltpu.VMEM((2, b, D), jnp.float32),
            pltpu.VMEM((zb, D), jnp.float32),
            pltpu.VMEM_SHARED((num_seg, D), jnp.float32),
            pltpu.SemaphoreType.DMA,
            pltpu.SemaphoreType.DMA,
            pltpu.SemaphoreType.DMA,
            pltpu.SemaphoreType.DMA,
        ],
    )
    def segsum(table_hbm, sd_hbm, out_hbm, sd, rows, zbuf, acc, g0, g1, i0, i1):
        c = lax.axis_index("c")
        s = lax.axis_index("s")
        nb = lax.select(c == 0, nb0, nb1)
        q0 = lax.select(c == 0, s * nb0, NS * nb0 + s * nb1)

        def work():

            pltpu.async_copy(sd_hbm.at[q0], sd.at[0], i0)
            pltpu.async_copy(sd_hbm.at[q0 + 1], sd.at[1], i1)

            _zero_fill(zbuf, zb)
            row0 = pl.multiple_of(s * zr, 8)
            for q in range(zr // zb):
                pltpu.async_copy(zbuf, acc.at[pl.ds(row0 + q * zb, zb)], g1)
            for q in range(zr // zb):
                pltpu.make_async_copy(zbuf, acc.at[pl.ds(row0, zb)], g1).wait()

            pltpu.make_async_copy(sd_hbm.at[q0], sd.at[0], i0).wait()
            pltpu.async_copy(table_hbm.at[sd.at[0, 0]], rows.at[0], g0)
            plsc.subcore_barrier()

            def body(i, carry):
                k0 = 2 * i
                pltpu.make_async_copy(sd_hbm.at[q0 + 1], sd.at[1], i1).wait()
                pltpu.async_copy(table_hbm.at[sd.at[1, 0]], rows.at[1], g1)
                pltpu.make_async_copy(
                    table_hbm.at[sd.at[0, 0]], rows.at[0], g0).wait()
                pltpu.sync_copy(rows.at[0], acc.at[sd.at[0, 1]], add=True)

                @pl.when(k0 + 2 < nb)
                def _():
                    pltpu.async_copy(sd_hbm.at[q0 + k0 + 2], sd.at[0], i0)

                pltpu.make_async_copy(
                    table_hbm.at[sd.at[1, 0]], rows.at[1], g1).wait()
                pltpu.sync_copy(rows.at[1], acc.at[sd.at[1, 1]], add=True)

                @pl.when(k0 + 3 < nb)
                def _():
                    pltpu.async_copy(sd_hbm.at[q0 + k0 + 3], sd.at[1], i1)

                @pl.when(k0 + 2 < nb)
                def _():
                    pltpu.make_async_copy(
                        sd_hbm.at[q0 + k0 + 2], sd.at[0], i0).wait()
                    pltpu.async_copy(table_hbm.at[sd.at[0, 0]], rows.at[0], g0)

                return carry

            lax.fori_loop(0, nb // 2, body, 0)
            plsc.subcore_barrier()

            base = pl.multiple_of(c * num_seg + row0, 8)
            pltpu.sync_copy(acc.at[pl.ds(row0, zr)], out_hbm.at[pl.ds(base, zr)])

        work()

    return segsum


BN = 1000


def _mlp_body(norm, p_ref, wa_ref, ba_ref, wb_ref, bb_ref, o_ref):
    x = p_ref[0]
    for q in range(1, p_ref.shape[0]):
        x = x + p_ref[q]
    y = jnp.maximum(
        jnp.dot(x, wa_ref[...], preferred_element_type=jnp.float32) + ba_ref[...], 0.0)
    z = jnp.maximum(
        jnp.dot(y, wb_ref[...], preferred_element_type=jnp.float32) + bb_ref[...], 0.0)
    if norm:
        nrm = jnp.sqrt(jnp.sum(z * z, axis=1, keepdims=True))
        z = z / jnp.maximum(nrm, 1e-12)
    o_ref[...] = z


def _mlp(p, wa, ba, wb, bb, norm):
    return pl.pallas_call(
        functools.partial(_mlp_body, norm),
        out_shape=jax.ShapeDtypeStruct((N, D), jnp.float32),
        grid=(N // BN,),
        in_specs=[
            pl.BlockSpec((p.shape[0], BN, D), lambda i: (0, i, 0)),
            pl.BlockSpec((D, D), lambda i: (0, 0)),
            pl.BlockSpec((1, D), lambda i: (0, 0)),
            pl.BlockSpec((D, D), lambda i: (0, 0)),
            pl.BlockSpec((1, D), lambda i: (0, 0)),
        ],
        out_specs=pl.BlockSpec((BN, D), lambda i: (i, 0)),
    )(p, wa, ba, wb, bb)


def _loss_body(psub_ref, idx_ref, o_ref):
    i = pl.program_id(0)
    sub = (psub_ref[:S] + psub_ref[S:]) * (1.0 / M)
    idx2d = idx_ref[0]
    ohT = jnp.where(
        lax.broadcasted_iota(jnp.int32, (S, idx2d.shape[1]), 0)
        == jnp.broadcast_to(idx2d, (S, idx2d.shape[1])), 1.0, 0.0)
    f = lax.dot_general(
        ohT, sub, (((0,), (0,)), ((), ())), preferred_element_type=jnp.float32)
    nrm = jnp.sqrt(jnp.sum(f * f, axis=1, keepdims=True))
    fn = f / jnp.maximum(nrm, 1e-12)
    logits = lax.dot_general(
        fn, fn, (((1,), (1,)), ((), ())), preferred_element_type=jnp.float32)
    logits = logits * (1.0 / TEMP)
    logits = logits - jnp.max(logits, axis=1, keepdims=True)
    r = lax.broadcasted_iota(jnp.int32, logits.shape, 0)
    cc = lax.broadcasted_iota(jnp.int32, logits.shape, 1)
    lmask = jnp.where(r == cc, 0.0, 1.0)
    pmask = jnp.where(lax.rem(r, 64) == lax.rem(cc, 64), 1.0, 0.0) * lmask
    el = jnp.exp(logits) * lmask
    denom = jnp.sum(el, axis=1, keepdims=True) + 1e-12
    lp = logits - jnp.log(denom)
    mlpp = jnp.sum(pmask * lp, axis=1) / jnp.sum(pmask, axis=1)
    bl = -jnp.sum(mlpp) / logits.shape[0]

    @pl.when(i == 0)
    def _():
        o_ref[...] = jnp.zeros((1, 1), jnp.float32)

    o_ref[...] += bl.reshape(1, 1)


def _loss(psub, cfidx3):
    nblk = cfidx3.shape[0]
    return pl.pallas_call(
        _loss_body,
        out_shape=jax.ShapeDtypeStruct((1, 1), jnp.float32),
        grid=(nblk,),
        in_specs=[
            pl.BlockSpec((NC * S, D), lambda i: (0, 0)),
            pl.BlockSpec((1, 1, cfidx3.shape[2]), lambda i: (i, 0, 0)),
        ],
        out_specs=pl.BlockSpec((1, 1), lambda i: (0, 0)),
    )(psub, cfidx3)


def kernel(seq1, adj, train_nodeSet, pathDict, bsz,
           W0a, b0a, W0b, b0b, W1a, b1a, W1b, b1b, W2a, b2a, W2b, b2b):
    npad = EPAD - E
    pad_src = jnp.arange(npad, dtype=jnp.int32) % N
    pad_dst = N + (jnp.arange(npad, dtype=jnp.int32) % 128)
    srcp = jnp.concatenate([adj[0], pad_src])
    dstp = jnp.concatenate([adj[1], pad_dst])
    sd_edges = jnp.stack([srcp.reshape(-1, EB), dstp.reshape(-1, EB)], axis=1)

    _segsum_edges = _make_segsum(NP, ENB, EB)
    _segsum_pool = _make_segsum(S, PNB, PB)

    h = seq1
    layers = ((W0a, b0a, W0b, b0b), (W1a, b1a, W1b, b1b), (W2a, b2a, W2b, b2b))
    for li, (wa, ba, wb, bb) in enumerate(layers):
        p = _segsum_edges(h, sd_edges).reshape(NC, NP, D)[:, :N]
        h = _mlp(p, wa, ba.reshape(1, D), wb, bb.reshape(1, D),
                 norm=(li == len(layers) - 1))

    sd_pool = jnp.stack(
        [train_nodeSet.reshape(-1, PB),
         jnp.repeat(jnp.arange(S, dtype=jnp.int32), M).reshape(-1, PB)], axis=1)
    psub = _segsum_pool(h, sd_pool)

    vi = jnp.concatenate(
        [pathDict, jnp.arange(S, dtype=pathDict.dtype)[:, None]], axis=1)
    cfidx3 = jnp.transpose(vi.reshape(4, 64, V + 1), (0, 2, 1)).reshape(4, 1, -1)
    lsum = _loss(psub, cfidx3.astype(jnp.int32))

    nb = S // bsz
    return lsum[0, 0] / nb


# speedup vs baseline: 1.1134x; 1.0671x over previous
import functools

import jax
import jax.numpy as jnp
from jax import lax
from jax.experimental import pallas as pl
from jax.experimental.pallas import tpu as pltpu
from jax.experimental.pallas import tpu_sc as plsc

N = 10000
E = 320000
D = 128
S = 256
M = 32
V = 3
TEMP = 0.5

NC = 2
NS = 16
NW = NC * NS

EB = 128
ENB = 80
EPAD = NW * ENB * EB
NP = 10240

PB = 128
PNB = 2


def _zero_fill(ref, rows):
    def body(i, carry):
        r = i // (D // 16)
        c = (i % (D // 16)) * 16
        ref[r, pl.ds(c, 16)] = jnp.zeros((16,), jnp.float32)
        return carry
    lax.fori_loop(0, rows * (D // 16), body, 0)


@functools.lru_cache(maxsize=None)
def _make_segsum(num_seg, nb, b, nb0=None):
    if nb0 is None:
        nb0 = nb
    nb1 = 2 * nb - nb0
    assert nb0 % 2 == 0 and nb1 % 2 == 0
    zr = num_seg // NS
    zb = zr if zr <= 64 else 64
    assert zr % zb == 0 and zr % 8 == 0
    mesh = plsc.VectorSubcoreMesh(core_axis_name="c", subcore_axis_name="s",
                                  num_cores=NC, num_subcores=NS)

    @functools.partial(
        pl.kernel,
        out_type=jax.ShapeDtypeStruct((NC * num_seg, D), jnp.float32),
        mesh=mesh,
        scratch_types=[
            pltpu.VMEM((2, 2, b), jnp.int32),
            pltpu.VMEM((2, b, D), jnp.float32),
            pltpu.VMEM((zb, D), jnp.float32),
            pltpu.VMEM_SHARED((num_seg, D), jnp.float32),
            pltpu.SemaphoreType.DMA,
            pltpu.SemaphoreType.DMA,
            pltpu.SemaphoreType.DMA,
            pltpu.SemaphoreType.DMA,
        ],
    )
    def segsum(table_hbm, adj_hbm, out_hbm, sd, rows, zbuf, acc, g0, g1, i0, i1):
        def idx_block(q):
            return adj_hbm.at[:, pl.ds(q * b, b)]
        c = lax.axis_index("c")
        s = lax.axis_index("s")
        nb = lax.select(c == 0, nb0, nb1)
        q0 = lax.select(c == 0, s * nb0, NS * nb0 + s * nb1)

        def work():

            pltpu.async_copy(idx_block(q0), sd.at[0], i0)
            pltpu.async_copy(idx_block(q0 + 1), sd.at[1], i1)

            _zero_fill(zbuf, zb)
            row0 = pl.multiple_of(s * zr, 8)
            for q in range(zr // zb):
                pltpu.async_copy(zbuf, acc.at[pl.ds(row0 + q * zb, zb)], g1)
            for q in range(zr // zb):
                pltpu.make_async_copy(zbuf, acc.at[pl.ds(row0, zb)], g1).wait()

            pltpu.make_async_copy(idx_block(q0), sd.at[0], i0).wait()
            pltpu.async_copy(table_hbm.at[sd.at[0, 0]], rows.at[0], g0)
            plsc.subcore_barrier()

            def body(i, carry):
                k0 = 2 * i
                pltpu.make_async_copy(idx_block(q0 + 1), sd.at[1], i1).wait()
                pltpu.async_copy(table_hbm.at[sd.at[1, 0]], rows.at[1], g1)
                pltpu.make_async_copy(
                    table_hbm.at[sd.at[0, 0]], rows.at[0], g0).wait()
                pltpu.sync_copy(rows.at[0], acc.at[sd.at[0, 1]], add=True)

                @pl.when(k0 + 2 < nb)
                def _():
                    pltpu.async_copy(idx_block(q0 + k0 + 2), sd.at[0], i0)

                pltpu.make_async_copy(
                    table_hbm.at[sd.at[1, 0]], rows.at[1], g1).wait()
                pltpu.sync_copy(rows.at[1], acc.at[sd.at[1, 1]], add=True)

                @pl.when(k0 + 3 < nb)
                def _():
                    pltpu.async_copy(idx_block(q0 + k0 + 3), sd.at[1], i1)

                @pl.when(k0 + 2 < nb)
                def _():
                    pltpu.make_async_copy(
                        idx_block(q0 + k0 + 2), sd.at[0], i0).wait()
                    pltpu.async_copy(table_hbm.at[sd.at[0, 0]], rows.at[0], g0)

                return carry

            lax.fori_loop(0, nb // 2, body, 0)
            plsc.subcore_barrier()

            base = pl.multiple_of(c * num_seg + row0, 8)
            pltpu.sync_copy(acc.at[pl.ds(row0, zr)], out_hbm.at[pl.ds(base, zr)])

        work()

    return segsum


BN = 1000


def _mlp_body(norm, p_ref, wa_ref, ba_ref, wb_ref, bb_ref, o_ref):
    x = p_ref[0]
    for q in range(1, p_ref.shape[0]):
        x = x + p_ref[q]
    y = jnp.maximum(
        jnp.dot(x, wa_ref[...], preferred_element_type=jnp.float32) + ba_ref[...], 0.0)
    z = jnp.maximum(
        jnp.dot(y, wb_ref[...], preferred_element_type=jnp.float32) + bb_ref[...], 0.0)
    if norm:
        nrm = jnp.sqrt(jnp.sum(z * z, axis=1, keepdims=True))
        z = z / jnp.maximum(nrm, 1e-12)
    o_ref[...] = z


def _mlp(p, wa, ba, wb, bb, norm):
    return pl.pallas_call(
        functools.partial(_mlp_body, norm),
        out_shape=jax.ShapeDtypeStruct((N, D), jnp.float32),
        grid=(N // BN,),
        in_specs=[
            pl.BlockSpec((p.shape[0], BN, D), lambda i: (0, i, 0)),
            pl.BlockSpec((D, D), lambda i: (0, 0)),
            pl.BlockSpec((1, D), lambda i: (0, 0)),
            pl.BlockSpec((D, D), lambda i: (0, 0)),
            pl.BlockSpec((1, D), lambda i: (0, 0)),
        ],
        out_specs=pl.BlockSpec((BN, D), lambda i: (i, 0)),
    )(p, wa, ba, wb, bb)


def _loss_body(psub_ref, idx_ref, o_ref):
    i = pl.program_id(0)
    sub = (psub_ref[:S] + psub_ref[S:]) * (1.0 / M)
    idx2d = idx_ref[0]
    ohT = jnp.where(
        lax.broadcasted_iota(jnp.int32, (S, idx2d.shape[1]), 0)
        == jnp.broadcast_to(idx2d, (S, idx2d.shape[1])), 1.0, 0.0)
    f = lax.dot_general(
        ohT, sub, (((0,), (0,)), ((), ())), preferred_element_type=jnp.float32)
    nrm = jnp.sqrt(jnp.sum(f * f, axis=1, keepdims=True))
    fn = f / jnp.maximum(nrm, 1e-12)
    logits = lax.dot_general(
        fn, fn, (((1,), (1,)), ((), ())), preferred_element_type=jnp.float32)
    logits = logits * (1.0 / TEMP)
    logits = logits - jnp.max(logits, axis=1, keepdims=True)
    r = lax.broadcasted_iota(jnp.int32, logits.shape, 0)
    cc = lax.broadcasted_iota(jnp.int32, logits.shape, 1)
    lmask = jnp.where(r == cc, 0.0, 1.0)
    pmask = jnp.where(lax.rem(r, 64) == lax.rem(cc, 64), 1.0, 0.0) * lmask
    el = jnp.exp(logits) * lmask
    denom = jnp.sum(el, axis=1, keepdims=True) + 1e-12
    lp = logits - jnp.log(denom)
    mlpp = jnp.sum(pmask * lp, axis=1) / jnp.sum(pmask, axis=1)
    bl = -jnp.sum(mlpp) / logits.shape[0]

    @pl.when(i == 0)
    def _():
        o_ref[...] = jnp.zeros((1, 1), jnp.float32)

    o_ref[...] += bl.reshape(1, 1)


def _loss(psub, cfidx3):
    nblk = cfidx3.shape[0]
    return pl.pallas_call(
        _loss_body,
        out_shape=jax.ShapeDtypeStruct((1, 1), jnp.float32),
        grid=(nblk,),
        in_specs=[
            pl.BlockSpec((NC * S, D), lambda i: (0, 0)),
            pl.BlockSpec((1, 1, cfidx3.shape[2]), lambda i: (i, 0, 0)),
        ],
        out_specs=pl.BlockSpec((1, 1), lambda i: (0, 0)),
    )(psub, cfidx3)


def kernel(seq1, adj, train_nodeSet, pathDict, bsz,
           W0a, b0a, W0b, b0b, W1a, b1a, W1b, b1b, W2a, b2a, W2b, b2b):
    npad = EPAD - E
    pad_src = jnp.arange(npad, dtype=jnp.int32) % N
    pad_dst = N + (jnp.arange(npad, dtype=jnp.int32) % 128)
    adjp = jnp.concatenate([adj, jnp.stack([pad_src, pad_dst])], axis=1)

    _segsum_edges = _make_segsum(NP, ENB, EB)
    _segsum_pool = _make_segsum(S, PNB, PB)

    h = seq1
    layers = ((W0a, b0a, W0b, b0b), (W1a, b1a, W1b, b1b), (W2a, b2a, W2b, b2b))
    for li, (wa, ba, wb, bb) in enumerate(layers):
        p = _segsum_edges(h, adjp).reshape(NC, NP, D)
        h = _mlp(p, wa, ba.reshape(1, D), wb, bb.reshape(1, D),
                 norm=(li == len(layers) - 1))

    adj_pool = jnp.stack(
        [train_nodeSet.reshape(-1),
         jnp.repeat(jnp.arange(S, dtype=jnp.int32), M)])
    psub = _segsum_pool(h, adj_pool)

    vi = jnp.concatenate(
        [pathDict, jnp.arange(S, dtype=pathDict.dtype)[:, None]], axis=1)
    cfidx3 = jnp.transpose(vi.reshape(4, 64, V + 1), (0, 2, 1)).reshape(4, 1, -1)
    lsum = _loss(psub, cfidx3.astype(jnp.int32))

    nb = S // bsz
    return lsum[0, 0] / nb
